# baseline (speedup 1.0000x reference)
import jax
import jax.numpy as jnp
from jax import lax
from jax.experimental import pallas as pl
from jax.experimental.pallas import tpu as pltpu
from jax.experimental.pallas import tpu_sc as plsc

N_PTS = 65536
BL = 2048
GRID = N_PTS // BL

_NSUB = 16
_PER = N_PTS // _NSUB
_CH = _PER // 16
_ROUNDS = 31
_SLOTS = _ROUNDS + 4


def _err_body(pred_ref, gt_ref, out_ref, bits_ref):
    d = jnp.abs(pred_ref[...] - gt_ref[...])
    e = jnp.sum(d, axis=(0, 1)) * jnp.float32(1.0 / 3.0)
    out_ref[...] = e
    bits_ref[...] = lax.bitcast_convert_type(e, jnp.int32)


def _lane_sum(x, dbuf):
    dbuf[pl.ds(0, 16)] = x
    dbuf[pl.ds(16, 16)] = x
    tot = x
    for j in range(1, 16):
        tot = tot + dbuf[pl.ds(j, 16)]
    return tot


def _lane_max(x, dbuf):
    dbuf[pl.ds(0, 16)] = x
    dbuf[pl.ds(16, 16)] = x
    tot = x
    for j in range(1, 16):
        tot = jnp.maximum(tot, dbuf[pl.ds(j, 16)])
    return tot


def _merge_slab(sh, r, stage):
    pltpu.sync_copy(sh.at[r], stage)
    g16 = stage[0]
    for i in range(1, _NSUB):
        g16 = g16 + stage[i]
    return g16


def _sc_select(err_hbm, bits_hbm, ratio_hbm, out_hbm,
               vals, bits, ratio_v, cbuf, stage, outbuf, dbuf_f, sh):
    cid = lax.axis_index("c")
    sid = lax.axis_index("s")
    pltpu.sync_copy(err_hbm.at[pl.ds(sid * _PER, _PER)], vals)
    pltpu.sync_copy(bits_hbm.at[pl.ds(sid * _PER, _PER)], bits)
    pltpu.sync_copy(ratio_hbm, ratio_v)
    k_vec = (ratio_v[...] * jnp.float32(N_PTS)).astype(jnp.int32)
    kf = k_vec.astype(jnp.float32)

    lo = jnp.zeros((16,), jnp.int32)
    hi = jnp.full((16,), 0x7F7FFFFF, jnp.int32)
    ones = jnp.full((16,), 1, jnp.int32)
    onesf = jnp.full((16,), 1.0, jnp.float32)
    for r in range(_ROUNDS):
        mid = lo + ((hi - lo + 1) >> 1)

        def cbody(i, acc, mid=mid):
            b = bits[pl.ds(i * 16, 16)]
            return acc + (ones + ((b - mid) >> 31)).astype(jnp.float32)

        cnt16 = lax.fori_loop(0, _CH, cbody, jnp.zeros((16,), jnp.float32))
        cbuf[...] = cnt16
        pltpu.sync_copy(cbuf, sh.at[r, sid])
        plsc.subcore_barrier()
        g16 = _merge_slab(sh, r, stage)
        cnt = _lane_sum(g16, dbuf_f)
        m = (cnt - kf).astype(jnp.int32) >> 31
        lo = (mid & ~m) | (lo & m)
        hi = (hi & ~m) | ((mid - 1) & m)

    t_bits = lo

    def fbody(i, carry):
        tot, sgt, cgt, mle = carry
        b = bits[pl.ds(i * 16, 16)]
        v = vals[pl.ds(i * 16, 16)]
        gt_f = (jnp.int32(0) - ((t_bits - b) >> 31)).astype(jnp.float32)
        le_f = (ones + ((t_bits - b) >> 31)).astype(jnp.float32)
        return (tot + v,
                sgt + v * gt_f,
                cgt + gt_f,
                jnp.maximum(mle, v * le_f))

    z = jnp.zeros((16,), jnp.float32)
    tot16, sgt16, cgt16, mle16 = lax.fori_loop(
        0, _CH, fbody, (z, z, z, z))

    cbuf[...] = tot16
    pltpu.sync_copy(cbuf, sh.at[_ROUNDS, sid])
    cbuf[...] = sgt16
    pltpu.sync_copy(cbuf, sh.at[_ROUNDS + 1, sid])
    cbuf[...] = cgt16
    pltpu.sync_copy(cbuf, sh.at[_ROUNDS + 2, sid])
    cbuf[...] = mle16
    pltpu.sync_copy(cbuf, sh.at[_ROUNDS + 3, sid])
    plsc.subcore_barrier()

    s_total = _lane_sum(_merge_slab(sh, _ROUNDS, stage), dbuf_f)
    s_gt = _lane_sum(_merge_slab(sh, _ROUNDS + 1, stage), dbuf_f)
    c_gt = _lane_sum(_merge_slab(sh, _ROUNDS + 2, stage), dbuf_f)
    pltpu.sync_copy(sh.at[_ROUNDS + 3], stage)
    mx16 = stage[0]
    for i in range(1, _NSUB):
        mx16 = jnp.maximum(mx16, stage[i])
    t_val = _lane_max(mx16, dbuf_f)
    s_topk = s_gt + (kf - c_gt) * t_val
    res = (s_total - s_topk) / (jnp.float32(N_PTS) - kf)
    outbuf[...] = res

    @pl.when(cid + sid == 0)
    def _():
        pltpu.sync_copy(outbuf, out_hbm)


def kernel(pred, gt, mask, penalize_ratio):
    del mask
    a = jnp.transpose(pred, (2, 1, 0))
    b = jnp.transpose(gt, (2, 1, 0))
    err, err_bits = pl.pallas_call(
        _err_body,
        grid=(GRID,),
        in_specs=[
            pl.BlockSpec((3, 49, BL), lambda i: (0, 0, i)),
            pl.BlockSpec((3, 49, BL), lambda i: (0, 0, i)),
        ],
        out_specs=[
            pl.BlockSpec((BL,), lambda i: (i,)),
            pl.BlockSpec((BL,), lambda i: (i,)),
        ],
        out_shape=[
            jax.ShapeDtypeStruct((N_PTS,), jnp.float32),
            jax.ShapeDtypeStruct((N_PTS,), jnp.int32),
        ],
    )(a, b)

    ratio16 = jnp.full((16,), jnp.asarray(penalize_ratio, jnp.float32))
    mesh = plsc.VectorSubcoreMesh(core_axis_name="c", subcore_axis_name="s")
    sel = pl.kernel(
        _sc_select,
        mesh=mesh,
        out_type=jax.ShapeDtypeStruct((16,), jnp.float32),
        scratch_types=[
            pltpu.VMEM((_PER,), jnp.float32),
            pltpu.VMEM((_PER,), jnp.int32),
            pltpu.VMEM((16,), jnp.float32),
            pltpu.VMEM((16,), jnp.float32),
            pltpu.VMEM((_NSUB, 16), jnp.float32),
            pltpu.VMEM((16,), jnp.float32),
            pltpu.VMEM((32,), jnp.float32),
            pltpu.VMEM_SHARED((_SLOTS, _NSUB, 16), jnp.float32),
        ],
    )
    res = sel(err, err_bits, ratio16)
    return res[0]


# speedup vs baseline: 1.3170x; 1.3170x over previous
import jax
import jax.numpy as jnp
from jax import lax
from jax.experimental import pallas as pl
from jax.experimental.pallas import tpu as pltpu
from jax.experimental.pallas import tpu_sc as plsc

N_PTS = 65536
BL = 2048
GRID = N_PTS // BL

_NSUB = 16
_PER = N_PTS // _NSUB
_CH = _PER // 16
_ROUNDS = 31
_SLOTS = _ROUNDS + 4


def _err_body(pred_ref, gt_ref, out_ref, bits_ref):
    d = jnp.abs(pred_ref[...] - gt_ref[...])
    e = jnp.sum(d, axis=(0, 1)) * jnp.float32(1.0 / 3.0)
    out_ref[...] = e
    bits_ref[...] = lax.bitcast_convert_type(e, jnp.int32)


def _lane_sum(x, dbuf):
    dbuf[pl.ds(0, 16)] = x
    dbuf[pl.ds(16, 16)] = x
    tot = x
    for j in range(1, 16):
        tot = tot + dbuf[pl.ds(j, 16)]
    return tot


def _lane_max(x, dbuf):
    dbuf[pl.ds(0, 16)] = x
    dbuf[pl.ds(16, 16)] = x
    tot = x
    for j in range(1, 16):
        tot = jnp.maximum(tot, dbuf[pl.ds(j, 16)])
    return tot


def _merge_slab(sh, r, stage):
    pltpu.sync_copy(sh.at[r], stage)
    g16 = stage[0]
    for i in range(1, _NSUB):
        g16 = g16 + stage[i]
    return g16


def _sc_select(err_hbm, bits_hbm, ratio_hbm, out_hbm,
               vals, bits, ratio_v, cbuf, stage, outbuf, dbuf_f, sh):
    cid = lax.axis_index("c")
    sid = lax.axis_index("s")
    pltpu.sync_copy(err_hbm.at[pl.ds(sid * _PER, _PER)], vals)
    pltpu.sync_copy(bits_hbm.at[pl.ds(sid * _PER, _PER)], bits)
    pltpu.sync_copy(ratio_hbm, ratio_v)
    k_vec = (ratio_v[...] * jnp.float32(N_PTS)).astype(jnp.int32)
    kf = k_vec.astype(jnp.float32)

    lo = jnp.zeros((16,), jnp.int32)
    hi = jnp.full((16,), 0x7F7FFFFF, jnp.int32)
    ones = jnp.full((16,), 1, jnp.int32)
    onesf = jnp.full((16,), 1.0, jnp.float32)
    for r in range(_ROUNDS):
        mid = lo + ((hi - lo + 1) >> 1)

        def cbody(i, acc, mid=mid):
            a0 = ones + ((bits[pl.ds(i * 64, 16)] - mid) >> 31)
            a1 = ones + ((bits[pl.ds(i * 64 + 16, 16)] - mid) >> 31)
            a2 = ones + ((bits[pl.ds(i * 64 + 32, 16)] - mid) >> 31)
            a3 = ones + ((bits[pl.ds(i * 64 + 48, 16)] - mid) >> 31)
            return acc + ((a0 + a1) + (a2 + a3)).astype(jnp.float32)

        cnt16 = lax.fori_loop(0, _CH // 4, cbody,
                              jnp.zeros((16,), jnp.float32))
        cbuf[...] = cnt16
        pltpu.sync_copy(cbuf, sh.at[r, sid])
        plsc.subcore_barrier()
        g16 = _merge_slab(sh, r, stage)
        cnt = _lane_sum(g16, dbuf_f)
        m = (cnt - kf).astype(jnp.int32) >> 31
        lo = (mid & ~m) | (lo & m)
        hi = (hi & ~m) | ((mid - 1) & m)

    t_bits = lo

    def fbody(i, carry):
        tot, sgt, cgt, mle = carry
        b = bits[pl.ds(i * 16, 16)]
        v = vals[pl.ds(i * 16, 16)]
        gt_f = (jnp.int32(0) - ((t_bits - b) >> 31)).astype(jnp.float32)
        le_f = (ones + ((t_bits - b) >> 31)).astype(jnp.float32)
        return (tot + v,
                sgt + v * gt_f,
                cgt + gt_f,
                jnp.maximum(mle, v * le_f))

    z = jnp.zeros((16,), jnp.float32)
    tot16, sgt16, cgt16, mle16 = lax.fori_loop(
        0, _CH, fbody, (z, z, z, z))

    cbuf[...] = tot16
    pltpu.sync_copy(cbuf, sh.at[_ROUNDS, sid])
    cbuf[...] = sgt16
    pltpu.sync_copy(cbuf, sh.at[_ROUNDS + 1, sid])
    cbuf[...] = cgt16
    pltpu.sync_copy(cbuf, sh.at[_ROUNDS + 2, sid])
    cbuf[...] = mle16
    pltpu.sync_copy(cbuf, sh.at[_ROUNDS + 3, sid])
    plsc.subcore_barrier()

    s_total = _lane_sum(_merge_slab(sh, _ROUNDS, stage), dbuf_f)
    s_gt = _lane_sum(_merge_slab(sh, _ROUNDS + 1, stage), dbuf_f)
    c_gt = _lane_sum(_merge_slab(sh, _ROUNDS + 2, stage), dbuf_f)
    pltpu.sync_copy(sh.at[_ROUNDS + 3], stage)
    mx16 = stage[0]
    for i in range(1, _NSUB):
        mx16 = jnp.maximum(mx16, stage[i])
    t_val = _lane_max(mx16, dbuf_f)
    s_topk = s_gt + (kf - c_gt) * t_val
    res = (s_total - s_topk) / (jnp.float32(N_PTS) - kf)
    outbuf[...] = res

    @pl.when(cid + sid == 0)
    def _():
        pltpu.sync_copy(outbuf, out_hbm)


def kernel(pred, gt, mask, penalize_ratio):
    del mask
    a = jnp.transpose(pred, (2, 1, 0))
    b = jnp.transpose(gt, (2, 1, 0))
    err, err_bits = pl.pallas_call(
        _err_body,
        grid=(GRID,),
        in_specs=[
            pl.BlockSpec((3, 49, BL), lambda i: (0, 0, i)),
            pl.BlockSpec((3, 49, BL), lambda i: (0, 0, i)),
        ],
        out_specs=[
            pl.BlockSpec((BL,), lambda i: (i,)),
            pl.BlockSpec((BL,), lambda i: (i,)),
        ],
        out_shape=[
            jax.ShapeDtypeStruct((N_PTS,), jnp.float32),
            jax.ShapeDtypeStruct((N_PTS,), jnp.int32),
        ],
    )(a, b)

    ratio16 = jnp.full((16,), jnp.asarray(penalize_ratio, jnp.float32))
    mesh = plsc.VectorSubcoreMesh(core_axis_name="c", subcore_axis_name="s")
    sel = pl.kernel(
        _sc_select,
        mesh=mesh,
        out_type=jax.ShapeDtypeStruct((16,), jnp.float32),
        scratch_types=[
            pltpu.VMEM((_PER,), jnp.float32),
            pltpu.VMEM((_PER,), jnp.int32),
            pltpu.VMEM((16,), jnp.float32),
            pltpu.VMEM((16,), jnp.float32),
            pltpu.VMEM((_NSUB, 16), jnp.float32),
            pltpu.VMEM((16,), jnp.float32),
            pltpu.VMEM((32,), jnp.float32),
            pltpu.VMEM_SHARED((_SLOTS, _NSUB, 16), jnp.float32),
        ],
    )
    res = sel(err, err_bits, ratio16)
    return res[0]


# speedup vs baseline: 1.3758x; 1.0446x over previous
import jax
import jax.numpy as jnp
from jax import lax
from jax.experimental import pallas as pl
from jax.experimental.pallas import tpu as pltpu
from jax.experimental.pallas import tpu_sc as plsc

N_PTS = 65536
BL = 2048
GRID = N_PTS // BL

_NSUB = 16
_PER = N_PTS // _NSUB
_CH = _PER // 16
_ROUNDS = 31
_SLOTS = _ROUNDS + 4


def _err_body(pred_ref, gt_ref, out_ref, bits_ref):
    d = jnp.abs(pred_ref[...] - gt_ref[...])
    e = jnp.sum(d, axis=(0, 1)) * jnp.float32(1.0 / 3.0)
    out_ref[...] = e
    bits_ref[...] = lax.bitcast_convert_type(e, jnp.int32)


def _lane_sum(x, dbuf):
    dbuf[pl.ds(0, 16)] = x
    dbuf[pl.ds(16, 16)] = x
    tot = x
    for j in range(1, 16):
        tot = tot + dbuf[pl.ds(j, 16)]
    return tot


def _lane_max(x, dbuf):
    dbuf[pl.ds(0, 16)] = x
    dbuf[pl.ds(16, 16)] = x
    tot = x
    for j in range(1, 16):
        tot = jnp.maximum(tot, dbuf[pl.ds(j, 16)])
    return tot


def _merge_slab(sh, r, stage):
    pltpu.sync_copy(sh.at[r], stage)
    g16 = stage[0]
    for i in range(1, _NSUB):
        g16 = g16 + stage[i]
    return g16


def _sc_select(err_hbm, bits_hbm, ratio_hbm, out_hbm,
               vals, bits, ratio_v, cbuf, stage, outbuf, dbuf_f, sh):
    cid = lax.axis_index("c")
    sid = lax.axis_index("s")
    pltpu.sync_copy(err_hbm.at[pl.ds(sid * _PER, _PER)], vals)
    pltpu.sync_copy(bits_hbm.at[pl.ds(sid * _PER, _PER)], bits)
    pltpu.sync_copy(ratio_hbm, ratio_v)
    k_vec = (ratio_v[...] * jnp.float32(N_PTS)).astype(jnp.int32)
    kf = k_vec.astype(jnp.float32)

    lo = jnp.zeros((16,), jnp.int32)
    hi = jnp.full((16,), 0x7F7FFFFF, jnp.int32)
    ones = jnp.full((16,), 1, jnp.int32)
    onesf = jnp.full((16,), 1.0, jnp.float32)
    for r in range(_ROUNDS):
        mid = lo + ((hi - lo + 1) >> 1)

        def cbody(i, acc, mid=mid):
            s01 = ((bits[pl.ds(i * 128, 16)] - mid) >> 31) + \
                  ((bits[pl.ds(i * 128 + 16, 16)] - mid) >> 31)
            s23 = ((bits[pl.ds(i * 128 + 32, 16)] - mid) >> 31) + \
                  ((bits[pl.ds(i * 128 + 48, 16)] - mid) >> 31)
            s45 = ((bits[pl.ds(i * 128 + 64, 16)] - mid) >> 31) + \
                  ((bits[pl.ds(i * 128 + 80, 16)] - mid) >> 31)
            s67 = ((bits[pl.ds(i * 128 + 96, 16)] - mid) >> 31) + \
                  ((bits[pl.ds(i * 128 + 112, 16)] - mid) >> 31)
            neg = (s01 + s23) + (s45 + s67)
            return acc + (jnp.full((16,), 8, jnp.int32) + neg).astype(jnp.float32)

        cnt16 = lax.fori_loop(0, _CH // 8, cbody,
                              jnp.zeros((16,), jnp.float32))
        cbuf[...] = cnt16
        pltpu.sync_copy(cbuf, sh.at[r, sid])
        plsc.subcore_barrier()
        g16 = _merge_slab(sh, r, stage)
        cnt = _lane_sum(g16, dbuf_f)
        m = (cnt - kf).astype(jnp.int32) >> 31
        lo = (mid & ~m) | (lo & m)
        hi = (hi & ~m) | ((mid - 1) & m)

    t_bits = lo

    def fbody(i, carry):
        tot, sgt, cgt, mle = carry
        b = bits[pl.ds(i * 16, 16)]
        v = vals[pl.ds(i * 16, 16)]
        gt_f = (jnp.int32(0) - ((t_bits - b) >> 31)).astype(jnp.float32)
        le_f = (ones + ((t_bits - b) >> 31)).astype(jnp.float32)
        return (tot + v,
                sgt + v * gt_f,
                cgt + gt_f,
                jnp.maximum(mle, v * le_f))

    z = jnp.zeros((16,), jnp.float32)
    tot16, sgt16, cgt16, mle16 = lax.fori_loop(
        0, _CH, fbody, (z, z, z, z))

    cbuf[...] = tot16
    pltpu.sync_copy(cbuf, sh.at[_ROUNDS, sid])
    cbuf[...] = sgt16
    pltpu.sync_copy(cbuf, sh.at[_ROUNDS + 1, sid])
    cbuf[...] = cgt16
    pltpu.sync_copy(cbuf, sh.at[_ROUNDS + 2, sid])
    cbuf[...] = mle16
    pltpu.sync_copy(cbuf, sh.at[_ROUNDS + 3, sid])
    plsc.subcore_barrier()

    s_total = _lane_sum(_merge_slab(sh, _ROUNDS, stage), dbuf_f)
    s_gt = _lane_sum(_merge_slab(sh, _ROUNDS + 1, stage), dbuf_f)
    c_gt = _lane_sum(_merge_slab(sh, _ROUNDS + 2, stage), dbuf_f)
    pltpu.sync_copy(sh.at[_ROUNDS + 3], stage)
    mx16 = stage[0]
    for i in range(1, _NSUB):
        mx16 = jnp.maximum(mx16, stage[i])
    t_val = _lane_max(mx16, dbuf_f)
    s_topk = s_gt + (kf - c_gt) * t_val
    res = (s_total - s_topk) / (jnp.float32(N_PTS) - kf)
    outbuf[...] = res

    @pl.when(cid + sid == 0)
    def _():
        pltpu.sync_copy(outbuf, out_hbm)


def kernel(pred, gt, mask, penalize_ratio):
    del mask
    a = jnp.transpose(pred, (2, 1, 0))
    b = jnp.transpose(gt, (2, 1, 0))
    err, err_bits = pl.pallas_call(
        _err_body,
        grid=(GRID,),
        in_specs=[
            pl.BlockSpec((3, 49, BL), lambda i: (0, 0, i)),
            pl.BlockSpec((3, 49, BL), lambda i: (0, 0, i)),
        ],
        out_specs=[
            pl.BlockSpec((BL,), lambda i: (i,)),
            pl.BlockSpec((BL,), lambda i: (i,)),
        ],
        out_shape=[
            jax.ShapeDtypeStruct((N_PTS,), jnp.float32),
            jax.ShapeDtypeStruct((N_PTS,), jnp.int32),
        ],
    )(a, b)

    ratio16 = jnp.full((16,), jnp.asarray(penalize_ratio, jnp.float32))
    mesh = plsc.VectorSubcoreMesh(core_axis_name="c", subcore_axis_name="s")
    sel = pl.kernel(
        _sc_select,
        mesh=mesh,
        out_type=jax.ShapeDtypeStruct((16,), jnp.float32),
        scratch_types=[
            pltpu.VMEM((_PER,), jnp.float32),
            pltpu.VMEM((_PER,), jnp.int32),
            pltpu.VMEM((16,), jnp.float32),
            pltpu.VMEM((16,), jnp.float32),
            pltpu.VMEM((_NSUB, 16), jnp.float32),
            pltpu.VMEM((16,), jnp.float32),
            pltpu.VMEM((32,), jnp.float32),
            pltpu.VMEM_SHARED((_SLOTS, _NSUB, 16), jnp.float32),
        ],
    )
    res = sel(err, err_bits, ratio16)
    return res[0]
